# baseline (speedup 1.0000x reference)
import functools

import jax
import jax.numpy as jnp
from jax import lax
from jax.experimental import pallas as pl
from jax.experimental.pallas import tpu as pltpu
from jax.experimental.pallas import tpu_sc as plsc

BATCH = 16384
EMBED_DIM = 32

_NC = 2
_NS = 16
_NW = _NC * _NS
_B_PER_W = BATCH // _NW
_CHUNK = 128
_NCHUNK = _B_PER_W // _CHUNK

_mesh = plsc.VectorSubcoreMesh(core_axis_name="c", subcore_axis_name="s")


@functools.partial(
    pl.kernel,
    out_type=jax.ShapeDtypeStruct((BATCH, EMBED_DIM), jnp.float32),
    mesh=_mesh,
    scratch_types=[
        pltpu.VMEM((_NCHUNK, _CHUNK), jnp.int32),
        pltpu.VMEM((_B_PER_W, EMBED_DIM), jnp.float32),
        pltpu.SemaphoreType.DMA,
    ],
    compiler_params=pltpu.CompilerParams(use_tc_tiling_on_sc=False),
)
def _gather_kernel(idx_hbm, table_hbm, out_hbm, idx_v, rows_v, sem):
    wid = lax.axis_index("s") * _NC + lax.axis_index("c")
    base = wid * _B_PER_W
    for j in range(_NCHUNK):
        pltpu.sync_copy(idx_hbm.at[pl.ds(base + j * _CHUNK, _CHUNK)],
                        idx_v.at[j])
    copies = []
    for j in range(_NCHUNK):
        copies.append(
            pltpu.async_copy(
                table_hbm.at[idx_v.at[j]],
                rows_v.at[pl.ds(j * _CHUNK, _CHUNK)],
                sem,
            ))
    for c in copies:
        c.wait()
    pltpu.sync_copy(rows_v, out_hbm.at[pl.ds(base, _B_PER_W)])


def kernel(idx, embedding):
    return _gather_kernel(idx.astype(jnp.int32), embedding)


# speedup vs baseline: 2.1107x; 2.1107x over previous
import functools

import jax
import jax.numpy as jnp
from jax import lax
from jax.experimental import pallas as pl
from jax.experimental.pallas import tpu as pltpu
from jax.experimental.pallas import tpu_sc as plsc

NUM_ITEMS = 100000
EMBED_DIM = 32
BATCH = 16384

_NC = 2
_NS = 16
_CHUNK = 4096
_NCHUNK = BATCH // _CHUNK
_LANES = 16

_mesh = plsc.VectorSubcoreMesh(core_axis_name="c", subcore_axis_name="s")


@functools.partial(
    pl.kernel,
    out_type=jax.ShapeDtypeStruct((EMBED_DIM, BATCH), jnp.float32),
    mesh=_mesh,
    scratch_types=[
        pltpu.VMEM((NUM_ITEMS,), jnp.float32),
        pltpu.VMEM((_CHUNK,), jnp.int32),
        pltpu.VMEM((_CHUNK,), jnp.float32),
    ],
    compiler_params=pltpu.CompilerParams(
        use_tc_tiling_on_sc=True, needs_layout_passes=False),
)
def _lookup_t_kernel(idx_hbm, tbl_hbm, out_hbm, row_v, idx_v, val_v):
    dim = lax.axis_index("s") * _NC + lax.axis_index("c")
    pltpu.sync_copy(tbl_hbm.at[dim], row_v)

    def gather16(k, _):
        iv = idx_v[pl.ds(k * _LANES, _LANES)]
        val_v[pl.ds(k * _LANES, _LANES)] = plsc.load_gather(row_v, [iv])
        return _

    for c in range(_NCHUNK):
        pltpu.sync_copy(idx_hbm.at[pl.ds(c * _CHUNK, _CHUNK)], idx_v)
        lax.fori_loop(0, _CHUNK // _LANES, gather16, 0, unroll=8)
        pltpu.sync_copy(val_v, out_hbm.at[dim, pl.ds(c * _CHUNK, _CHUNK)])


def kernel(idx, embedding):
    out_t = _lookup_t_kernel(idx.astype(jnp.int32), embedding.T)
    return out_t.T


# speedup vs baseline: 2.3051x; 1.0921x over previous
import functools

import jax
import jax.numpy as jnp
from jax import lax
from jax.experimental import pallas as pl
from jax.experimental.pallas import tpu as pltpu
from jax.experimental.pallas import tpu_sc as plsc

NUM_ITEMS = 100000
EMBED_DIM = 32
BATCH = 16384

_NC = 2
_NS = 16
_CHUNK = 4096
_NCHUNK = BATCH // _CHUNK
_LANES = 16

_mesh = plsc.VectorSubcoreMesh(core_axis_name="c", subcore_axis_name="s")


@functools.partial(
    pl.kernel,
    out_type=jax.ShapeDtypeStruct((EMBED_DIM, BATCH), jnp.float32),
    mesh=_mesh,
    scratch_types=[
        pltpu.VMEM((NUM_ITEMS,), jnp.float32),
        pltpu.VMEM((_NCHUNK, _CHUNK), jnp.int32),
        pltpu.VMEM((2, _CHUNK), jnp.float32),
        pltpu.SemaphoreType.DMA,
        pltpu.SemaphoreType.DMA,
        pltpu.SemaphoreType.DMA,
    ],
    compiler_params=pltpu.CompilerParams(
        use_tc_tiling_on_sc=True, needs_layout_passes=False),
)
def _lookup_t_kernel(idx_hbm, tbl_hbm, out_hbm, row_v, idx_v, val_v,
                     sem_row, sem_idx, sem_out):
    dim = lax.axis_index("s") * _NC + lax.axis_index("c")
    row_cp = pltpu.async_copy(tbl_hbm.at[dim], row_v, sem_row)
    idx_cps = [
        pltpu.async_copy(idx_hbm.at[pl.ds(c * _CHUNK, _CHUNK)],
                         idx_v.at[c], sem_idx)
        for c in range(_NCHUNK)
    ]
    row_cp.wait()

    out_cps = []
    for c in range(_NCHUNK):
        idx_cps[c].wait()
        if c >= 2:
            out_cps[c - 2].wait()
        buf = c % 2

        def gather16(k, _, c=c, buf=buf):
            iv = idx_v[c, pl.ds(k * _LANES, _LANES)]
            val_v[buf, pl.ds(k * _LANES, _LANES)] = plsc.load_gather(
                row_v, [iv])
            return _

        lax.fori_loop(0, _CHUNK // _LANES, gather16, 0, unroll=8)
        out_cps.append(
            pltpu.async_copy(val_v.at[buf],
                             out_hbm.at[dim, pl.ds(c * _CHUNK, _CHUNK)],
                             sem_out))
    out_cps[-2].wait()
    out_cps[-1].wait()


def kernel(idx, embedding):
    out_t = _lookup_t_kernel(idx.astype(jnp.int32), embedding.T)
    return out_t.T
